# Initial kernel scaffold; baseline (speedup 1.0000x reference)
import jax
import jax.numpy as jnp
from jax.experimental import pallas as pl


def kernel(x, w_patch, b_patch, w_fc_t, b_fc, w_proj_t, b_proj):
    raise NotImplementedError("write your pallas kernel here")


# speedup vs baseline: 1.1469x; 1.1469x over previous
import jax
import jax.numpy as jnp
from jax.experimental import pallas as pl
from jax.experimental.pallas import tpu as pltpu

_PATCH = 4
_NUM_CLASSES = 16


def kernel(x, w_patch, b_patch, w_fc_t, b_fc, w_proj_t, b_proj):
    B, _, T, F = x.shape
    patch = _PATCH
    Hp, Wp = F // patch, T // patch
    K = patch * patch
    G = B * Wp
    NC = _NUM_CLASSES
    C2 = w_patch.shape[1]
    C = C2 // 2

    xb = jnp.transpose(x, (0, 1, 3, 2))
    p = xb.reshape(B, 1, Hp, patch, Wp, patch)
    p = jnp.transpose(p, (2, 0, 4, 1, 3, 5))
    patches = p.reshape(Hp * G, K)

    def _sed_kernel(p_ref, wp_ref, bp_ref, wfc_ref, bfc_ref, wpr_ref,
                    bpr_ref, o_ref):
        j = pl.program_id(0)

        emb = jnp.maximum(
            jnp.dot(p_ref[...], wp_ref[...], preferred_element_type=jnp.float32)
            + bp_ref[...], 0.0)

        xacc = emb[0:G, :]
        for h in range(1, Hp):
            xacc = xacc + emb[h * G:(h + 1) * G, :]
        xt = xacc * (1.0 / Hp)

        zrow = jnp.zeros((1, C), jnp.float32)
        x_prev = jnp.concatenate([zrow, xt[:-1, :]], axis=0)
        x_next = jnp.concatenate([xt[1:, :], zrow], axis=0)
        t_idx = jax.lax.broadcasted_iota(jnp.int32, (G, C), 0) % Wp
        first = t_idx == 0
        last = t_idx == Wp - 1
        x1 = jnp.maximum(xt, jnp.maximum(jnp.where(first, -jnp.inf, x_prev),
                                         jnp.where(last, -jnp.inf, x_next)))
        x2 = (xt + jnp.where(first, 0.0, x_prev)
              + jnp.where(last, 0.0, x_next)) * (1.0 / 3.0)
        xs = x1 + x2

        y = jnp.maximum(
            jnp.dot(xs, wfc_ref[...], preferred_element_type=jnp.float32)
            + bfc_ref[...], 0.0)

        z = (jnp.dot(y, wpr_ref[...], preferred_element_type=jnp.float32)
             + bpr_ref[...])
        att_all = jnp.tanh(z[:, :2 * NC])
        cla_all = jax.nn.sigmoid(z[:, 2 * NC:])
        is0 = j == 0
        att = jnp.where(is0, att_all[:, :NC], att_all[:, NC:2 * NC])
        cla = jnp.where(is0, cla_all[:, :NC], cla_all[:, NC:2 * NC])

        preds = []
        for b in range(B):
            a_b = att[b * Wp:(b + 1) * Wp, :]
            c_b = cla[b * Wp:(b + 1) * Wp, :]
            m = jnp.max(a_b, axis=0, keepdims=True)
            e = jnp.exp(a_b - m)
            norm_att = e * pl.reciprocal(jnp.sum(e, axis=0, keepdims=True),
                                         approx=True)
            clip = jnp.sum(norm_att * c_b, axis=0, keepdims=True)
            maxframe = jnp.max(c_b, axis=0, keepdims=True)
            preds.append(0.5 * (clip + maxframe))
        pred = jnp.concatenate(preds, axis=0)

        wmix = jnp.where(is0, 0.3, 0.7)
        o_ref[...] = (wmix * pred)[None, :, :]

    partial = pl.pallas_call(
        _sed_kernel,
        out_shape=jax.ShapeDtypeStruct((2, B, NC), jnp.float32),
        grid_spec=pltpu.PrefetchScalarGridSpec(
            num_scalar_prefetch=0,
            grid=(2,),
            in_specs=[
                pl.BlockSpec((Hp * G, K), lambda j: (0, 0)),
                pl.BlockSpec((K, C), lambda j: (0, j)),
                pl.BlockSpec((1, C), lambda j: (0, j)),
                pl.BlockSpec((C, C), lambda j: (j, j)),
                pl.BlockSpec((1, C), lambda j: (0, j)),
                pl.BlockSpec((C, 4 * NC), lambda j: (j, 0)),
                pl.BlockSpec((1, 4 * NC), lambda j: (0, 0)),
            ],
            out_specs=pl.BlockSpec((1, B, NC), lambda j: (j, 0, 0)),
        ),
        compiler_params=pltpu.CompilerParams(
            dimension_semantics=("parallel",)),
    )(patches, w_patch, b_patch, w_fc_t, b_fc, w_proj_t, b_proj)

    pred = partial[0] + partial[1]
    return pred, pred


# speedup vs baseline: 1.6440x; 1.4334x over previous
import jax
import jax.numpy as jnp
from jax.experimental import pallas as pl
from jax.experimental.pallas import tpu as pltpu

_PATCH = 4
_NUM_CLASSES = 16


def kernel(x, w_patch, b_patch, w_fc_t, b_fc, w_proj_t, b_proj):
    B, _, T, F = x.shape
    patch = _PATCH
    Hp, Wp = F // patch, T // patch
    K = patch * patch
    G = B * Wp
    NC = _NUM_CLASSES
    C2 = w_patch.shape[1]
    C = C2 // 2
    BT = B * T
    R = Hp * G

    def _sed_kernel(x_ref, wp_ref, bp_ref, wfc_ref, bfc_ref, wpr_ref,
                    bpr_ref, o_ref, patches_s):
        j = pl.program_id(0)

        @pl.when(j == 0)
        def _build_patches():
            X = x_ref[...].reshape(BT, F)
            r_i = jax.lax.broadcasted_iota(jnp.int32, (R, BT), 0)
            c_i = jax.lax.broadcasted_iota(jnp.int32, (R, BT), 1)
            rf_i = jax.lax.broadcasted_iota(jnp.int32, (R, F), 0)
            cf_i = jax.lax.broadcasted_iota(jnp.int32, (R, F), 1)
            msk = (cf_i // patch) == (rf_i // G)
            rk = jax.lax.broadcasted_iota(jnp.int32, (F, K), 0)
            kk = jax.lax.broadcasted_iota(jnp.int32, (F, K), 1)
            acc = jnp.zeros((R, K), jnp.float32)
            for pt in range(patch):
                sel = (c_i == (r_i % G) * patch + pt).astype(jnp.float32)
                a = jnp.dot(sel, X, preferred_element_type=jnp.float32)
                a = jnp.where(msk, a, 0.0)
                cc = (kk == (rk % patch) * patch + pt).astype(jnp.float32)
                acc = acc + jnp.dot(a, cc, preferred_element_type=jnp.float32)
            patches_s[...] = acc

        emb = jnp.maximum(
            jnp.dot(patches_s[...], wp_ref[...],
                    preferred_element_type=jnp.float32)
            + bp_ref[...], 0.0)

        xacc = emb[0:G, :]
        for h in range(1, Hp):
            xacc = xacc + emb[h * G:(h + 1) * G, :]
        xt = xacc * (1.0 / Hp)

        zrow = jnp.zeros((1, C), jnp.float32)
        x_prev = jnp.concatenate([zrow, xt[:-1, :]], axis=0)
        x_next = jnp.concatenate([xt[1:, :], zrow], axis=0)
        t_idx = jax.lax.broadcasted_iota(jnp.int32, (G, C), 0) % Wp
        first = t_idx == 0
        last = t_idx == Wp - 1
        x1 = jnp.maximum(xt, jnp.maximum(jnp.where(first, -jnp.inf, x_prev),
                                         jnp.where(last, -jnp.inf, x_next)))
        x2 = (xt + jnp.where(first, 0.0, x_prev)
              + jnp.where(last, 0.0, x_next)) * (1.0 / 3.0)
        xs = x1 + x2

        y = jnp.maximum(
            jnp.dot(xs, wfc_ref[...], preferred_element_type=jnp.float32)
            + bfc_ref[...], 0.0)

        z = (jnp.dot(y, wpr_ref[...], preferred_element_type=jnp.float32)
             + bpr_ref[...])
        att_all = jnp.tanh(z[:, :2 * NC])
        cla_all = jax.nn.sigmoid(z[:, 2 * NC:])
        is0 = j == 0
        att = jnp.where(is0, att_all[:, :NC], att_all[:, NC:2 * NC])
        cla = jnp.where(is0, cla_all[:, :NC], cla_all[:, NC:2 * NC])

        preds = []
        for b in range(B):
            a_b = att[b * Wp:(b + 1) * Wp, :]
            c_b = cla[b * Wp:(b + 1) * Wp, :]
            m = jnp.max(a_b, axis=0, keepdims=True)
            e = jnp.exp(a_b - m)
            norm_att = e * pl.reciprocal(jnp.sum(e, axis=0, keepdims=True),
                                         approx=True)
            clip = jnp.sum(norm_att * c_b, axis=0, keepdims=True)
            maxframe = jnp.max(c_b, axis=0, keepdims=True)
            preds.append(0.5 * (clip + maxframe))
        pred = jnp.concatenate(preds, axis=0)

        @pl.when(j == 0)
        def _init_out():
            o_ref[...] = 0.3 * pred

        @pl.when(j > 0)
        def _acc_out():
            o_ref[...] = o_ref[...] + 0.7 * pred

    pred = pl.pallas_call(
        _sed_kernel,
        out_shape=jax.ShapeDtypeStruct((B, NC), jnp.float32),
        grid_spec=pltpu.PrefetchScalarGridSpec(
            num_scalar_prefetch=0,
            grid=(2,),
            in_specs=[
                pl.BlockSpec((B, 1, T, F), lambda j: (0, 0, 0, 0)),
                pl.BlockSpec((K, C), lambda j: (0, j)),
                pl.BlockSpec((1, C), lambda j: (0, j)),
                pl.BlockSpec((C, C), lambda j: (j, j)),
                pl.BlockSpec((1, C), lambda j: (0, j)),
                pl.BlockSpec((C, 4 * NC), lambda j: (j, 0)),
                pl.BlockSpec((1, 4 * NC), lambda j: (0, 0)),
            ],
            out_specs=pl.BlockSpec((B, NC), lambda j: (0, 0)),
            scratch_shapes=[pltpu.VMEM((R, K), jnp.float32)],
        ),
        compiler_params=pltpu.CompilerParams(
            dimension_semantics=("arbitrary",)),
    )(x, w_patch, b_patch, w_fc_t, b_fc, w_proj_t, b_proj)

    return pred, pred
